# Initial kernel scaffold; baseline (speedup 1.0000x reference)
import jax
import jax.numpy as jnp
from jax.experimental import pallas as pl


def kernel(feat, edge_index, W1, b1, W_mu, b_mu, W_ls, b_ls):
    raise NotImplementedError("write your pallas kernel here")


# speedup vs baseline: 4.5559x; 4.5559x over previous
import functools

import jax
import jax.numpy as jnp
from jax import lax
from jax.experimental import pallas as pl
from jax.experimental.pallas import tpu as pltpu
from jax.experimental.pallas import tpu_sc as plsc

NC = 2
NS = 16
NW = NC * NS
LANES = 16

CHUNK = 128
DEG_W = 128


def _sc_mesh():
    return plsc.VectorSubcoreMesh(
        core_axis_name="c", subcore_axis_name="s", num_cores=NC, num_subcores=NS
    )


def _make_degree_kernel(nbins, ch):
    stripe = nbins // NS

    @functools.partial(
        pl.kernel,
        out_type=jax.ShapeDtypeStruct((NC, nbins, DEG_W), jnp.float32),
        mesh=_sc_mesh(),
        scratch_types=[
            pltpu.VMEM((ch, CHUNK), jnp.int32),
            pltpu.VMEM((CHUNK, DEG_W), jnp.float32),
            pltpu.VMEM((8, DEG_W), jnp.float32),
            pltpu.VMEM_SHARED((nbins, DEG_W), jnp.float32),
        ],
    )
    def deg_kernel(idx_hbm, const_hbm, out_hbm, idxv, ones_v, zrow, acc):
        c = lax.axis_index("c")
        s = lax.axis_index("s")

        pltpu.sync_copy(idx_hbm.at[c, s], idxv)
        pltpu.sync_copy(const_hbm.at[pl.ds(0, CHUNK)], ones_v)
        pltpu.sync_copy(const_hbm.at[pl.ds(CHUNK, 8)], zrow)

        base = s * stripe

        def zero_body(t, _):
            pltpu.sync_copy(zrow, acc.at[pl.ds(base + t * 8, 8)])
            return _
        lax.fori_loop(0, stripe // 8, zero_body, None)

        plsc.subcore_barrier()

        def hist_body(j, _):
            pltpu.sync_copy(ones_v, acc.at[idxv.at[j]], add=True)
            return _
        lax.fori_loop(0, ch, hist_body, None)

        plsc.subcore_barrier()

        sl = pl.ds(base, stripe)
        pltpu.sync_copy(acc.at[sl], out_hbm.at[c, sl])

    return deg_kernel


def _make_agg_kernel(n_rows, nacc, ch, feat_w):
    stripe = nacc // NS

    @functools.partial(
        pl.kernel,
        out_type=jax.ShapeDtypeStruct((NC, nacc, feat_w), jnp.float32),
        mesh=_sc_mesh(),
        scratch_types=[
            pltpu.VMEM((ch, CHUNK), jnp.int32),
            pltpu.VMEM((ch, CHUNK), jnp.int32),
            pltpu.VMEM((CHUNK, feat_w), jnp.float32),
            pltpu.VMEM((8, feat_w), jnp.float32),
            pltpu.VMEM_SHARED((nacc, feat_w), jnp.float32),
            pltpu.SemaphoreType.DMA,
        ],
    )
    def agg_kernel(table_hbm, src_hbm, dst_hbm, out_hbm, srcv, dstv, rows,
                   zrow, acc, sem):
        c = lax.axis_index("c")
        s = lax.axis_index("s")
        wid = s * NC + c

        pltpu.sync_copy(src_hbm.at[wid], srcv)
        pltpu.sync_copy(dst_hbm.at[wid], dstv)

        zvec = jnp.zeros((LANES,), jnp.float32)
        for r in range(8):
            for k in range(feat_w // LANES):
                zrow[r, pl.ds(k * LANES, LANES)] = zvec

        base = s * stripe

        def zero_body(t, _):
            pltpu.sync_copy(zrow, acc.at[pl.ds(base + t * 8, 8)])
            return _
        lax.fori_loop(0, stripe // 8, zero_body, None)

        plsc.subcore_barrier()

        def edge_body(j, _):
            pltpu.async_copy(table_hbm.at[srcv.at[j]], rows, sem).wait()
            pltpu.sync_copy(rows, acc.at[dstv.at[j]], add=True)
            return _
        lax.fori_loop(0, ch, edge_body, None)

        plsc.subcore_barrier()

        sl = pl.ds(base, stripe)
        pltpu.sync_copy(acc.at[sl], out_hbm.at[c, sl])

    return agg_kernel


def _norms_from_degs(degs_ref, kind):
    d = jnp.sum(degs_ref[kind], axis=-1)
    return lax.rsqrt(jnp.maximum(d, jnp.float32(1.0)))


def _tc_scale_body(feat_ref, degs_ref, out_ref):
    nsrc = _norms_from_degs(degs_ref, 0)
    out_ref[...] = feat_ref[...] * nsrc[:, None]


def _tc_layer1_body(p0_ref, p1_ref, degs_ref, w_ref, b_ref, out_ref):
    ndst = _norms_from_degs(degs_ref, 1)
    nsrc = _norms_from_degs(degs_ref, 0)
    agg = (p0_ref[...] + p1_ref[...]) * ndst[:, None]
    hpre = jnp.dot(agg, w_ref[...], preferred_element_type=jnp.float32)
    hrelu = jnp.maximum(hpre + b_ref[...], 0.0)
    out_ref[...] = hrelu * nsrc[:, None]


def _tc_heads_body(p0_ref, p1_ref, degs_ref, wmu_ref, bmu_ref, wls_ref,
                   bls_ref, noise_ref, out_ref):
    ndst = _norms_from_degs(degs_ref, 1)
    rst = (p0_ref[...] + p1_ref[...]) * ndst[:, None]
    mu = jnp.dot(rst, wmu_ref[...], preferred_element_type=jnp.float32)
    ls = jnp.dot(rst, wls_ref[...], preferred_element_type=jnp.float32)
    out_ref[...] = (mu + bmu_ref[...]
                    + noise_ref[...] * jnp.exp(ls + bls_ref[...]))


def kernel(feat, edge_index, W1, b1, W_mu, b_mu, W_ls, b_ls):
    n, f = feat.shape
    h = W1.shape[1]
    e = edge_index.shape[1]

    ch = -(-e // (NW * CHUNK))
    ep = NW * CHUNK * ch
    pad = ep - e

    nacc = -(-(n + 1) // (NS * 8)) * (NS * 8)
    rb = 1000
    grid = (n // rb,)

    src = edge_index[0]
    dst = edge_index[1]
    i32 = jnp.int32
    dst_h = jnp.concatenate([dst, jnp.full((pad,), n, i32)]).reshape(NW, ch, CHUNK)
    src_g = jnp.concatenate([src, jnp.zeros((pad,), i32)]).reshape(NW, ch, CHUNK)

    ch2 = -(-e // (NS * CHUNK))
    pad2 = NS * CHUNK * ch2 - e
    hist_idx = jnp.stack([
        jnp.concatenate([src, jnp.full((pad2,), n, i32)]).reshape(NS, ch2, CHUNK),
        jnp.concatenate([dst, jnp.full((pad2,), n, i32)]).reshape(NS, ch2, CHUNK),
    ])

    noise = jax.random.uniform(jax.random.key(1), (n, h), dtype=feat.dtype)

    const_rows = jnp.concatenate([
        jnp.tile(jax.nn.one_hot(0, DEG_W, dtype=jnp.float32)[None], (CHUNK, 1)),
        jnp.zeros((8, DEG_W), jnp.float32),
    ])

    degs = _make_degree_kernel(nacc, ch2)(hist_idx, const_rows)

    degs_spec = pl.BlockSpec((NC, rb, DEG_W), lambda i: (0, i, 0))
    mat_spec = pl.BlockSpec((rb, f), lambda i: (i, 0))
    w_spec = pl.BlockSpec((f, h), lambda i: (0, 0))
    b_spec = pl.BlockSpec((1, h), lambda i: (0, 0))

    table1 = pl.pallas_call(
        _tc_scale_body,
        grid=grid,
        in_specs=[mat_spec, degs_spec],
        out_specs=mat_spec,
        out_shape=jax.ShapeDtypeStruct((n, f), jnp.float32),
    )(feat, degs)

    agg_kernel = _make_agg_kernel(n, nacc, ch, f)
    parts1 = agg_kernel(table1, src_g, dst_h)

    table2 = pl.pallas_call(
        _tc_layer1_body,
        grid=grid,
        in_specs=[mat_spec, mat_spec, degs_spec, w_spec, b_spec],
        out_specs=pl.BlockSpec((rb, h), lambda i: (i, 0)),
        out_shape=jax.ShapeDtypeStruct((n, h), jnp.float32),
    )(parts1[0], parts1[1], degs, W1, b1.reshape(1, h))

    parts2 = agg_kernel(table2, src_g, dst_h)

    z = pl.pallas_call(
        _tc_heads_body,
        grid=grid,
        in_specs=[mat_spec, mat_spec, degs_spec, w_spec, b_spec, w_spec,
                  b_spec, pl.BlockSpec((rb, h), lambda i: (i, 0))],
        out_specs=pl.BlockSpec((rb, h), lambda i: (i, 0)),
        out_shape=jax.ShapeDtypeStruct((n, h), jnp.float32),
    )(parts2[0], parts2[1], degs, W_mu, b_mu.reshape(1, h), W_ls,
      b_ls.reshape(1, h), noise)

    return z
